# Initial kernel scaffold; baseline (speedup 1.0000x reference)
import functools

import jax
import jax.numpy as jnp
from jax import lax
from jax.experimental import pallas as pl
from jax.experimental.pallas import tpu as pltpu
from jax.experimental.pallas import tpu_sc as plsc

B, N, E = 8, 1024, 32768
NODE_DIM, EDGE_DIM, HIDDEN, HEADS = 128, 16, 128, 4
HEAD_DIM = HIDDEN // HEADS

NSUB = 16
NCORE = 2
EPT = E // NSUB
CHUNK = 128
NCHUNK = EPT // CHUNK
ROWS = N // NSUB
BPC = B // NCORE


def _prep_nodes_body(nf_ref, wnt_ref, asd_ref, h_ref, sd_ref):
    h = jnp.dot(nf_ref[0], wnt_ref[...], preferred_element_type=jnp.float32)
    h_ref[0] = h
    sd_ref[0] = jnp.dot(h, asd_ref[...], preferred_element_type=jnp.float32)


def _prep_edges_body(ea_ref, wet_ref, ae_ref, e_ref, el_ref):
    e = jnp.dot(ea_ref[0], wet_ref[...], preferred_element_type=jnp.float32)
    e_ref[0] = e
    el_ref[0] = jnp.dot(e, ae_ref[...], preferred_element_type=jnp.float32)


def _finish_body(agg_ref, den_ref, nf_ref, g_ref, b_ref, o_ref):
    den = den_ref[0]
    inv = 1.0 / jnp.where(den > 0, den, 1.0)
    invr = jnp.reshape(
        jnp.broadcast_to(inv[:, :, None], (N, HEADS, HEAD_DIM)), (N, HIDDEN))
    res = agg_ref[0] * invr + nf_ref[0]
    mean = jnp.mean(res, axis=1, keepdims=True)
    xc = res - mean
    var = jnp.mean(xc * xc, axis=1, keepdims=True)
    y = xc * lax.rsqrt(var + 1e-5) * g_ref[...] + b_ref[...]
    o_ref[0] = jnp.where(y > 0, y, jnp.exp(y) - 1.0)


_sc_mesh = plsc.VectorSubcoreMesh(core_axis_name="c", subcore_axis_name="s")


@functools.partial(
    pl.kernel,
    out_type=(
        jax.ShapeDtypeStruct((B, N, HIDDEN), jnp.float32),
        jax.ShapeDtypeStruct((B, HEADS, N), jnp.float32),
    ),
    mesh=_sc_mesh,
    scratch_types=[
        pltpu.VMEM((NCHUNK, CHUNK), jnp.int32),
        pltpu.VMEM((NCHUNK, CHUNK), jnp.int32),
        pltpu.VMEM((CHUNK,), jnp.int32),
        pltpu.VMEM((N, 8), jnp.float32),
        pltpu.VMEM((EPT, HEADS), jnp.float32),
        pltpu.VMEM((HEADS, EPT), jnp.float32),
        pltpu.VMEM((HEADS, N), jnp.float32),
        pltpu.VMEM((CHUNK, HIDDEN), jnp.float32),
        pltpu.VMEM((CHUNK, HIDDEN), jnp.float32),
        pltpu.VMEM((ROWS, HIDDEN), jnp.float32),
        pltpu.VMEM((NSUB, HEADS, ROWS), jnp.float32),
        pltpu.VMEM((HEADS, ROWS), jnp.float32),
        pltpu.VMEM_SHARED((N, HIDDEN), jnp.float32),
        pltpu.VMEM_SHARED((NSUB, HEADS, N), jnp.float32),
        pltpu.SemaphoreType.DMA,
    ],
)
def _sc_gat(h_hbm, sd_hbm, e_hbm, el_hbm, src_hbm, dst_hbm,
            agg_hbm, den_hbm,
            src_c_v, dst_c_v, gidx_v, sd_v, el_v, w_v, den_v, e_v, hs_v,
            z_v, red_v, dsum_v, agg_sh, den_all_sh, sem):
    cid = lax.axis_index("c")
    sid = lax.axis_index("s")

    pltpu.sync_copy(src_hbm.at[sid], src_c_v)
    pltpu.sync_copy(dst_hbm.at[sid], dst_c_v)

    def _zz(i, _):
        for j in range(HIDDEN // 16):
            z_v[i, pl.ds(j * 16, 16)] = jnp.zeros((16,), jnp.float32)
        return 0
    lax.fori_loop(0, ROWS, _zz, 0)

    def batch_body(bl, _):
        b = cid * BPC + bl

        pltpu.sync_copy(sd_hbm.at[b], sd_v)
        pltpu.sync_copy(el_hbm.at[b, pl.ds(sid * EPT, EPT)], el_v)

        def _zd(i, _):
            for h in range(HEADS):
                den_v[h, pl.ds(i * 16, 16)] = jnp.zeros((16,), jnp.float32)
            return 0
        lax.fori_loop(0, N // 16, _zd, 0)
        pltpu.sync_copy(z_v, agg_sh.at[pl.ds(sid * ROWS, ROWS)])
        plsc.subcore_barrier()

        def phase_a(g, _):
            c = g // (CHUNK // 16)
            o = (g % (CHUNK // 16)) * 16
            src16 = src_c_v[c, pl.ds(o, 16)]
            dst16 = dst_c_v[c, pl.ds(o, 16)]
            k16 = g * 16 + lax.iota(jnp.int32, 16)
            for h in range(HEADS):
                hh = jnp.full((16,), h, jnp.int32)
                sv = plsc.load_gather(sd_v, [src16, hh])
                dv = plsc.load_gather(sd_v, [dst16, jnp.full((16,), HEADS + h, jnp.int32)])
                ev = plsc.load_gather(el_v, [k16, hh])
                l = sv + dv + ev
                l = jnp.where(l >= 0, l, l * jnp.float32(0.2))
                w = jnp.exp(l)
                w_v[h, pl.ds(g * 16, 16)] = w
                plsc.addupdate_scatter(den_v, [hh, dst16], w)
            return 0
        lax.fori_loop(0, EPT // 16, phase_a, 0)

        def chunk_body(cb, _):
            pltpu.sync_copy(e_hbm.at[b, pl.ds(sid * EPT + cb * CHUNK, CHUNK)], e_v)
            base = b * N
            for j in range(CHUNK // 16):
                gidx_v[pl.ds(j * 16, 16)] = src_c_v[cb, pl.ds(j * 16, 16)] + base
            pltpu.async_copy(h_hbm.at[gidx_v], hs_v, sem).wait()

            def edge_body(k, _):
                for h in range(HEADS):
                    ws = w_v[h, cb * CHUNK + k]
                    wb = jnp.full((16,), ws)
                    for j2 in range(HEAD_DIM // 16):
                        col = h * HEAD_DIM + j2 * 16
                        m = (hs_v[k, pl.ds(col, 16)] + e_v[k, pl.ds(col, 16)]) * wb
                        hs_v[k, pl.ds(col, 16)] = m
                return 0
            lax.fori_loop(0, CHUNK, edge_body, 0)

            pltpu.sync_copy(hs_v, agg_sh.at[dst_c_v.at[cb]], add=True)
            return 0
        lax.fori_loop(0, NCHUNK, chunk_body, 0)

        pltpu.sync_copy(den_v, den_all_sh.at[sid])
        plsc.subcore_barrier()

        pltpu.sync_copy(agg_sh.at[pl.ds(sid * ROWS, ROWS)],
                        agg_hbm.at[b, pl.ds(sid * ROWS, ROWS)])
        pltpu.sync_copy(den_all_sh.at[:, :, pl.ds(sid * ROWS, ROWS)], red_v)
        for h in range(HEADS):
            for j in range(ROWS // 16):
                acc = red_v[0, h, pl.ds(j * 16, 16)]
                for t in range(1, NSUB):
                    acc = acc + red_v[t, h, pl.ds(j * 16, 16)]
                dsum_v[h, pl.ds(j * 16, 16)] = acc
        pltpu.sync_copy(dsum_v, den_hbm.at[b, :, pl.ds(sid * ROWS, ROWS)])
        plsc.subcore_barrier()
        return 0

    lax.fori_loop(0, BPC, batch_body, 0)


def kernel(node_feat, edge_index, edge_attr, W_node, W_edge,
           att_src, att_dst, att_edge, ln_gamma, ln_beta):
    f32 = jnp.float32
    eye = jnp.eye(HEADS, dtype=f32)
    a_src = (eye[:, None, :] * att_src[:, :, None]).reshape(HIDDEN, HEADS)
    a_dst = (eye[:, None, :] * att_dst[:, :, None]).reshape(HIDDEN, HEADS)
    a_edge = (eye[:, None, :] * att_edge[:, :, None]).reshape(HIDDEN, HEADS)
    a_sd = jnp.concatenate([a_src, a_dst], axis=1)

    h, sd = pl.pallas_call(
        _prep_nodes_body,
        grid=(B,),
        in_specs=[
            pl.BlockSpec((1, N, NODE_DIM), lambda b: (b, 0, 0)),
            pl.BlockSpec((NODE_DIM, HIDDEN), lambda b: (0, 0)),
            pl.BlockSpec((HIDDEN, 2 * HEADS), lambda b: (0, 0)),
        ],
        out_specs=[
            pl.BlockSpec((1, N, HIDDEN), lambda b: (b, 0, 0)),
            pl.BlockSpec((1, N, 2 * HEADS), lambda b: (b, 0, 0)),
        ],
        out_shape=[
            jax.ShapeDtypeStruct((B, N, HIDDEN), f32),
            jax.ShapeDtypeStruct((B, N, 2 * HEADS), f32),
        ],
    )(node_feat, W_node.T, a_sd)

    ECH = 4096
    e, el = pl.pallas_call(
        _prep_edges_body,
        grid=(B, E // ECH),
        in_specs=[
            pl.BlockSpec((1, ECH, EDGE_DIM), lambda b, c: (b, c, 0)),
            pl.BlockSpec((EDGE_DIM, HIDDEN), lambda b, c: (0, 0)),
            pl.BlockSpec((HIDDEN, HEADS), lambda b, c: (0, 0)),
        ],
        out_specs=[
            pl.BlockSpec((1, ECH, HIDDEN), lambda b, c: (b, c, 0)),
            pl.BlockSpec((1, ECH, HEADS), lambda b, c: (b, c, 0)),
        ],
        out_shape=[
            jax.ShapeDtypeStruct((B, E, HIDDEN), f32),
            jax.ShapeDtypeStruct((B, E, HEADS), f32),
        ],
    )(edge_attr, W_edge.T, a_edge)

    src_r = edge_index[0].reshape(NSUB, NCHUNK, CHUNK)
    dst_r = edge_index[1].reshape(NSUB, NCHUNK, CHUNK)
    h_flat = h.reshape(B * N, HIDDEN)

    agg, den = _sc_gat(h_flat, sd, e, el, src_r, dst_r)
    den_t = jnp.transpose(den, (0, 2, 1))

    out = pl.pallas_call(
        _finish_body,
        grid=(B,),
        in_specs=[
            pl.BlockSpec((1, N, HIDDEN), lambda b: (b, 0, 0)),
            pl.BlockSpec((1, N, HEADS), lambda b: (b, 0, 0)),
            pl.BlockSpec((1, N, HIDDEN), lambda b: (b, 0, 0)),
            pl.BlockSpec((1, HIDDEN), lambda b: (0, 0)),
            pl.BlockSpec((1, HIDDEN), lambda b: (0, 0)),
        ],
        out_specs=pl.BlockSpec((1, N, HIDDEN), lambda b: (b, 0, 0)),
        out_shape=jax.ShapeDtypeStruct((B, N, HIDDEN), f32),
    )(agg, den_t, node_feat, ln_gamma.reshape(1, HIDDEN), ln_beta.reshape(1, HIDDEN))
    return out


# speedup vs baseline: 13.2295x; 13.2295x over previous
import functools

import jax
import jax.numpy as jnp
from jax import lax
from jax.experimental import pallas as pl
from jax.experimental.pallas import tpu as pltpu
from jax.experimental.pallas import tpu_sc as plsc

B, N, E = 8, 1024, 32768
NODE_DIM, EDGE_DIM, HIDDEN, HEADS = 128, 16, 128, 4
HEAD_DIM = HIDDEN // HEADS

NSUB = 16
NCORE = 2
EPT = E // NSUB
CHUNK = 128
NCHUNK = EPT // CHUNK
ROWS = N // NSUB
BPC = B // NCORE
DTOT = HEADS * N
DSL = DTOT // NSUB


def _prep_nodes_body(nf_ref, wnt_ref, asd_ref, h_ref, sd_ref):
    h = jnp.dot(nf_ref[0], wnt_ref[...], preferred_element_type=jnp.float32)
    h_ref[0] = h
    sd_ref[0] = jnp.dot(h, asd_ref[...], preferred_element_type=jnp.float32)


def _prep_edges_body(ea_ref, wet_ref, ae_ref, e_ref, el_ref):
    e = jnp.dot(ea_ref[0], wet_ref[...], preferred_element_type=jnp.float32)
    e_ref[0] = e
    el_ref[0] = jnp.dot(e, ae_ref[...], preferred_element_type=jnp.float32)


def _finish_body(agg_ref, den_ref, nf_ref, g_ref, b_ref, o_ref):
    den = den_ref[0]
    inv = 1.0 / jnp.where(den > 0, den, 1.0)
    invr = jnp.reshape(
        jnp.broadcast_to(inv[:, :, None], (N, HEADS, HEAD_DIM)), (N, HIDDEN))
    res = agg_ref[0] * invr + nf_ref[0]
    mean = jnp.mean(res, axis=1, keepdims=True)
    xc = res - mean
    var = jnp.mean(xc * xc, axis=1, keepdims=True)
    y = xc * lax.rsqrt(var + 1e-5) * g_ref[...] + b_ref[...]
    o_ref[0] = jnp.where(y > 0, y, jnp.exp(y) - 1.0)


_sc_mesh = plsc.VectorSubcoreMesh(core_axis_name="c", subcore_axis_name="s")


@functools.partial(
    pl.kernel,
    out_type=(
        jax.ShapeDtypeStruct((B, N, HIDDEN), jnp.float32),
        jax.ShapeDtypeStruct((B, DTOT), jnp.float32),
    ),
    mesh=_sc_mesh,
    compiler_params=pltpu.CompilerParams(needs_layout_passes=False),
    scratch_types=[
        pltpu.VMEM((NCHUNK, CHUNK), jnp.int32),
        pltpu.VMEM((NCHUNK, CHUNK), jnp.int32),
        pltpu.VMEM((CHUNK,), jnp.int32),
        pltpu.VMEM((CHUNK,), jnp.int32),
        pltpu.VMEM((N * 8,), jnp.float32),
        pltpu.VMEM((EPT * HEADS,), jnp.float32),
        pltpu.VMEM((EPT * HEADS + 16,), jnp.float32),
        pltpu.VMEM((DTOT,), jnp.float32),
        pltpu.VMEM((NSUB * DSL,), jnp.float32),
        pltpu.VMEM((DSL,), jnp.float32),
        pltpu.VMEM((CHUNK, HIDDEN), jnp.float32),
        pltpu.VMEM((CHUNK, HIDDEN), jnp.float32),
        pltpu.VMEM((ROWS, HIDDEN), jnp.float32),
        pltpu.VMEM_SHARED((N, HIDDEN), jnp.float32),
        pltpu.VMEM_SHARED((NSUB * DTOT,), jnp.float32),
        pltpu.SemaphoreType.DMA,
    ],
)
def _sc_gat(h_hbm, sd_hbm, e_hbm, el_hbm, src_hbm, dst_hbm,
            agg_hbm, den_hbm,
            src_c_v, dst_c_v, gidx_v, didx_v, sd_v, el_v, w_v, den_v,
            red_v, dsum_v, e_v, hg_v, z_v, agg_sh, den_all_sh, sem):
    cid = lax.axis_index("c")
    sid = lax.axis_index("s")

    pltpu.sync_copy(src_hbm.at[sid], src_c_v)
    pltpu.sync_copy(dst_hbm.at[sid], dst_c_v)

    def _zz(i, _):
        for j in range(HIDDEN // 16):
            z_v[i, pl.ds(j * 16, 16)] = jnp.zeros((16,), jnp.float32)
        return 0
    lax.fori_loop(0, ROWS, _zz, 0)

    def batch_body(bl, _):
        b = cid * BPC + bl

        pltpu.sync_copy(sd_hbm.at[b], sd_v)
        pltpu.sync_copy(el_hbm.at[b, pl.ds(sid * EPT * HEADS, EPT * HEADS)], el_v)

        def _zd(i, _):
            den_v[pl.ds(i * 16, 16)] = jnp.zeros((16,), jnp.float32)
            return 0
        lax.fori_loop(0, DTOT // 16, _zd, 0)
        pltpu.sync_copy(z_v, agg_sh.at[pl.ds(sid * ROWS, ROWS)])
        plsc.subcore_barrier()

        def phase_a(g, _):
            c = g // (CHUNK // 16)
            o = (g % (CHUNK // 16)) * 16
            src16 = src_c_v[c, pl.ds(o, 16)]
            dst16 = dst_c_v[c, pl.ds(o, 16)]
            k16 = g * 16 + lax.iota(jnp.int32, 16)
            for h in range(HEADS):
                sv = plsc.load_gather(sd_v, [src16 * 8 + h])
                dv = plsc.load_gather(sd_v, [dst16 * 8 + (HEADS + h)])
                ev = plsc.load_gather(el_v, [k16 * HEADS + h])
                l = sv + dv + ev
                l = jnp.where(l >= 0, l, l * jnp.float32(0.2))
                w = jnp.exp(l)
                plsc.store_scatter(w_v, [k16 * HEADS + h], w)
                plsc.addupdate_scatter(den_v, [dst16 + h * N], w)
            return 0
        lax.fori_loop(0, EPT // 16, phase_a, 0)

        def chunk_body(cb, _):
            pltpu.sync_copy(e_hbm.at[b, pl.ds(sid * EPT + cb * CHUNK, CHUNK)], e_v)
            base = b * N
            for j in range(CHUNK // 16):
                gidx_v[pl.ds(j * 16, 16)] = src_c_v[cb, pl.ds(j * 16, 16)] + base
                didx_v[pl.ds(j * 16, 16)] = dst_c_v[cb, pl.ds(j * 16, 16)]
            pltpu.async_copy(h_hbm.at[gidx_v], hg_v, sem).wait()

            def edge_body(k, _):
                wrow = w_v[pl.ds((cb * CHUNK + k) * HEADS, 16)]
                for h in range(HEADS):
                    wb = jnp.full((16,), wrow[h])
                    for j2 in range(HEAD_DIM // 16):
                        col = h * HEAD_DIM + j2 * 16
                        m = (hg_v[k, pl.ds(col, 16)] + e_v[k, pl.ds(col, 16)]) * wb
                        hg_v[k, pl.ds(col, 16)] = m
                return 0
            lax.fori_loop(0, CHUNK, edge_body, 0)

            pltpu.sync_copy(hg_v, agg_sh.at[didx_v], add=True)
            return 0
        lax.fori_loop(0, NCHUNK, chunk_body, 0)

        pltpu.sync_copy(den_v, den_all_sh.at[pl.ds(sid * DTOT, DTOT)])
        plsc.subcore_barrier()

        pltpu.sync_copy(agg_sh.at[pl.ds(sid * ROWS, ROWS)],
                        agg_hbm.at[b, pl.ds(sid * ROWS, ROWS)])
        for t in range(NSUB):
            pltpu.sync_copy(den_all_sh.at[pl.ds(t * DTOT + sid * DSL, DSL)],
                            red_v.at[pl.ds(t * DSL, DSL)])
        for j in range(DSL // 16):
            acc = red_v[pl.ds(j * 16, 16)]
            for t in range(1, NSUB):
                acc = acc + red_v[pl.ds(t * DSL + j * 16, 16)]
            dsum_v[pl.ds(j * 16, 16)] = acc
        pltpu.sync_copy(dsum_v, den_hbm.at[b, pl.ds(sid * DSL, DSL)])
        plsc.subcore_barrier()
        return 0

    lax.fori_loop(0, BPC, batch_body, 0)


def kernel(node_feat, edge_index, edge_attr, W_node, W_edge,
           att_src, att_dst, att_edge, ln_gamma, ln_beta):
    f32 = jnp.float32
    eye = jnp.eye(HEADS, dtype=f32)
    a_src = (eye[:, None, :] * att_src[:, :, None]).reshape(HIDDEN, HEADS)
    a_dst = (eye[:, None, :] * att_dst[:, :, None]).reshape(HIDDEN, HEADS)
    a_edge = (eye[:, None, :] * att_edge[:, :, None]).reshape(HIDDEN, HEADS)
    a_sd = jnp.concatenate([a_src, a_dst], axis=1)

    h, sd = pl.pallas_call(
        _prep_nodes_body,
        grid=(B,),
        in_specs=[
            pl.BlockSpec((1, N, NODE_DIM), lambda b: (b, 0, 0)),
            pl.BlockSpec((NODE_DIM, HIDDEN), lambda b: (0, 0)),
            pl.BlockSpec((HIDDEN, 2 * HEADS), lambda b: (0, 0)),
        ],
        out_specs=[
            pl.BlockSpec((1, N, HIDDEN), lambda b: (b, 0, 0)),
            pl.BlockSpec((1, N, 2 * HEADS), lambda b: (b, 0, 0)),
        ],
        out_shape=[
            jax.ShapeDtypeStruct((B, N, HIDDEN), f32),
            jax.ShapeDtypeStruct((B, N, 2 * HEADS), f32),
        ],
    )(node_feat, W_node.T, a_sd)

    ECH = 4096
    e, el = pl.pallas_call(
        _prep_edges_body,
        grid=(B, E // ECH),
        in_specs=[
            pl.BlockSpec((1, ECH, EDGE_DIM), lambda b, c: (b, c, 0)),
            pl.BlockSpec((EDGE_DIM, HIDDEN), lambda b, c: (0, 0)),
            pl.BlockSpec((HIDDEN, HEADS), lambda b, c: (0, 0)),
        ],
        out_specs=[
            pl.BlockSpec((1, ECH, HIDDEN), lambda b, c: (b, c, 0)),
            pl.BlockSpec((1, ECH, HEADS), lambda b, c: (b, c, 0)),
        ],
        out_shape=[
            jax.ShapeDtypeStruct((B, E, HIDDEN), f32),
            jax.ShapeDtypeStruct((B, E, HEADS), f32),
        ],
    )(edge_attr, W_edge.T, a_edge)

    src_r = edge_index[0].reshape(NSUB, NCHUNK, CHUNK)
    dst_r = edge_index[1].reshape(NSUB, NCHUNK, CHUNK)
    h_flat = h.reshape(B * N, HIDDEN)

    agg, den = _sc_gat(h_flat, sd.reshape(B, N * 2 * HEADS),
                       e, el.reshape(B, E * HEADS), src_r, dst_r)
    den_t = jnp.transpose(den.reshape(B, HEADS, N), (0, 2, 1))

    out = pl.pallas_call(
        _finish_body,
        grid=(B,),
        in_specs=[
            pl.BlockSpec((1, N, HIDDEN), lambda b: (b, 0, 0)),
            pl.BlockSpec((1, N, HEADS), lambda b: (b, 0, 0)),
            pl.BlockSpec((1, N, HIDDEN), lambda b: (b, 0, 0)),
            pl.BlockSpec((1, HIDDEN), lambda b: (0, 0)),
            pl.BlockSpec((1, HIDDEN), lambda b: (0, 0)),
        ],
        out_specs=pl.BlockSpec((1, N, HIDDEN), lambda b: (b, 0, 0)),
        out_shape=jax.ShapeDtypeStruct((B, N, HIDDEN), f32),
    )(agg, den_t, node_feat, ln_gamma.reshape(1, HIDDEN), ln_beta.reshape(1, HIDDEN))
    return out


# speedup vs baseline: 19.0346x; 1.4388x over previous
import functools

import jax
import jax.numpy as jnp
from jax import lax
from jax.experimental import pallas as pl
from jax.experimental.pallas import tpu as pltpu
from jax.experimental.pallas import tpu_sc as plsc

B, N, E = 8, 1024, 32768
NODE_DIM, EDGE_DIM, HIDDEN, HEADS = 128, 16, 128, 4
HEAD_DIM = HIDDEN // HEADS

NSUB = 16
NCORE = 2
EPT = E // NSUB
CHUNK = 128
NCHUNK = EPT // CHUNK
ROWS = N // NSUB
BPC = B // NCORE
DTOT = HEADS * N
DSL = DTOT // NSUB


def _prep_nodes_body(nf_ref, wnt_ref, asd_ref, h_ref, sd_ref):
    h = jnp.dot(nf_ref[0], wnt_ref[...], preferred_element_type=jnp.float32)
    h_ref[...] = h
    sdb = jnp.dot(h, asd_ref[...], preferred_element_type=jnp.float32)
    sd_ref[0] = sdb.T


def _prep_edges_body(ea_ref, wet_ref, ae_ref, e_ref, el_ref):
    e = jnp.dot(ea_ref[0], wet_ref[...], preferred_element_type=jnp.float32)
    e_ref[0] = e
    el = jnp.dot(e, ae_ref[...], preferred_element_type=jnp.float32)
    el_ref[0] = el.T


def _finish_body(agg_ref, den_ref, nf_ref, g_ref, b_ref, o_ref):
    den = den_ref[0, :HEADS].T
    inv = 1.0 / jnp.where(den > 0, den, 1.0)
    invr = jnp.reshape(
        jnp.broadcast_to(inv[:, :, None], (N, HEADS, HEAD_DIM)), (N, HIDDEN))
    res = agg_ref[0] * invr + nf_ref[0]
    mean = jnp.mean(res, axis=1, keepdims=True)
    xc = res - mean
    var = jnp.mean(xc * xc, axis=1, keepdims=True)
    y = xc * lax.rsqrt(var + 1e-5) * g_ref[...] + b_ref[...]
    o_ref[0] = jnp.where(y > 0, y, jnp.exp(y) - 1.0)


_sc_mesh = plsc.VectorSubcoreMesh(core_axis_name="c", subcore_axis_name="s")


@functools.partial(
    pl.kernel,
    out_type=(
        jax.ShapeDtypeStruct((B, N, HIDDEN), jnp.float32),
        jax.ShapeDtypeStruct((B, 8, N), jnp.float32),
    ),
    mesh=_sc_mesh,
    compiler_params=pltpu.CompilerParams(needs_layout_passes=False),
    scratch_types=[
        pltpu.VMEM((NCHUNK, CHUNK), jnp.int32),
        pltpu.VMEM((NCHUNK, CHUNK), jnp.int32),
        pltpu.VMEM((CHUNK,), jnp.int32),
        pltpu.VMEM((CHUNK,), jnp.int32),
        pltpu.VMEM((N * 8,), jnp.float32),
        pltpu.VMEM((EPT * HEADS,), jnp.float32),
        pltpu.VMEM((EPT * HEADS + 16,), jnp.float32),
        pltpu.VMEM((DTOT,), jnp.float32),
        pltpu.VMEM((NSUB * DSL,), jnp.float32),
        pltpu.VMEM((DSL,), jnp.float32),
        pltpu.VMEM((CHUNK, HIDDEN), jnp.float32),
        pltpu.VMEM((CHUNK, HIDDEN), jnp.float32),
        pltpu.VMEM((ROWS, HIDDEN), jnp.float32),
        pltpu.VMEM_SHARED((N, HIDDEN), jnp.float32),
        pltpu.VMEM_SHARED((NSUB * DTOT,), jnp.float32),
        pltpu.SemaphoreType.DMA,
    ],
)
def _sc_gat(h_hbm, sd_hbm, e_hbm, el_hbm, src_hbm, dst_hbm,
            agg_hbm, den_hbm,
            src_c_v, dst_c_v, gidx_v, didx_v, sd_v, el_v, w_v, den_v,
            red_v, dsum_v, e_v, hg_v, z_v, agg_sh, den_all_sh, sem):
    cid = lax.axis_index("c")
    sid = lax.axis_index("s")

    pltpu.sync_copy(src_hbm.at[sid], src_c_v)
    pltpu.sync_copy(dst_hbm.at[sid], dst_c_v)

    def _zz(i, _):
        for j in range(HIDDEN // 16):
            z_v[i, pl.ds(j * 16, 16)] = jnp.zeros((16,), jnp.float32)
        return 0
    lax.fori_loop(0, ROWS, _zz, 0)

    def batch_body(bl, _):
        b = cid * BPC + bl

        pltpu.sync_copy(sd_hbm.at[b], sd_v)
        for h in range(HEADS):
            pltpu.sync_copy(el_hbm.at[b, h, pl.ds(sid * EPT, EPT)],
                            el_v.at[pl.ds(h * EPT, EPT)])

        def _zd(i, _):
            den_v[pl.ds(i * 16, 16)] = jnp.zeros((16,), jnp.float32)
            return 0
        lax.fori_loop(0, DTOT // 16, _zd, 0)
        pltpu.sync_copy(z_v, agg_sh.at[pl.ds(sid * ROWS, ROWS)])
        plsc.subcore_barrier()

        def phase_a(g, _):
            c = g // (CHUNK // 16)
            o = (g % (CHUNK // 16)) * 16
            src16 = src_c_v[c, pl.ds(o, 16)]
            dst16 = dst_c_v[c, pl.ds(o, 16)]
            k16 = g * 16 + lax.iota(jnp.int32, 16)
            for h in range(HEADS):
                sv = plsc.load_gather(sd_v, [src16 + h * N])
                dv = plsc.load_gather(sd_v, [dst16 + (HEADS + h) * N])
                ev = el_v[pl.ds(h * EPT + g * 16, 16)]
                l = sv + dv + ev
                l = jnp.where(l >= 0, l, l * jnp.float32(0.2))
                w = jnp.exp(l)
                plsc.store_scatter(w_v, [k16 * HEADS + h], w)
                plsc.addupdate_scatter(den_v, [dst16 + h * N], w)
            return 0
        lax.fori_loop(0, EPT // 16, phase_a, 0)

        def chunk_body(cb, _):
            pltpu.sync_copy(e_hbm.at[b, pl.ds(sid * EPT + cb * CHUNK, CHUNK)], e_v)
            base = b * N
            for j in range(CHUNK // 16):
                gidx_v[pl.ds(j * 16, 16)] = src_c_v[cb, pl.ds(j * 16, 16)] + base
                didx_v[pl.ds(j * 16, 16)] = dst_c_v[cb, pl.ds(j * 16, 16)]
            pltpu.async_copy(h_hbm.at[gidx_v], hg_v, sem).wait()

            def edge_body(k, _):
                wrow = w_v[pl.ds((cb * CHUNK + k) * HEADS, 16)]
                for h in range(HEADS):
                    wb = jnp.full((16,), wrow[h])
                    for j2 in range(HEAD_DIM // 16):
                        col = h * HEAD_DIM + j2 * 16
                        m = (hg_v[k, pl.ds(col, 16)] + e_v[k, pl.ds(col, 16)]) * wb
                        hg_v[k, pl.ds(col, 16)] = m
                return 0
            lax.fori_loop(0, CHUNK, edge_body, 0)

            pltpu.sync_copy(hg_v, agg_sh.at[didx_v], add=True)
            return 0
        lax.fori_loop(0, NCHUNK, chunk_body, 0)

        pltpu.sync_copy(den_v, den_all_sh.at[pl.ds(sid * DTOT, DTOT)])
        plsc.subcore_barrier()

        pltpu.sync_copy(agg_sh.at[pl.ds(sid * ROWS, ROWS)],
                        agg_hbm.at[b, pl.ds(sid * ROWS, ROWS)])
        for t in range(NSUB):
            pltpu.sync_copy(den_all_sh.at[pl.ds(t * DTOT + sid * DSL, DSL)],
                            red_v.at[pl.ds(t * DSL, DSL)])
        for j in range(DSL // 16):
            acc = red_v[pl.ds(j * 16, 16)]
            for t in range(1, NSUB):
                acc = acc + red_v[pl.ds(t * DSL + j * 16, 16)]
            dsum_v[pl.ds(j * 16, 16)] = acc
        pltpu.sync_copy(dsum_v,
                        den_hbm.at[b, sid // (N // DSL),
                                   pl.ds((sid % (N // DSL)) * DSL, DSL)])
        plsc.subcore_barrier()
        return 0

    lax.fori_loop(0, BPC, batch_body, 0)


def kernel(node_feat, edge_index, edge_attr, W_node, W_edge,
           att_src, att_dst, att_edge, ln_gamma, ln_beta):
    f32 = jnp.float32
    eye = jnp.eye(HEADS, dtype=f32)
    a_src = (eye[:, None, :] * att_src[:, :, None]).reshape(HIDDEN, HEADS)
    a_dst = (eye[:, None, :] * att_dst[:, :, None]).reshape(HIDDEN, HEADS)
    a_edge = (eye[:, None, :] * att_edge[:, :, None]).reshape(HIDDEN, HEADS)
    a_sd = jnp.concatenate([a_src, a_dst], axis=1)

    h, sd = pl.pallas_call(
        _prep_nodes_body,
        grid=(B,),
        in_specs=[
            pl.BlockSpec((1, N, NODE_DIM), lambda b: (b, 0, 0)),
            pl.BlockSpec((NODE_DIM, HIDDEN), lambda b: (0, 0)),
            pl.BlockSpec((HIDDEN, 2 * HEADS), lambda b: (0, 0)),
        ],
        out_specs=[
            pl.BlockSpec((N, HIDDEN), lambda b: (b, 0)),
            pl.BlockSpec((1, 2 * HEADS, N), lambda b: (b, 0, 0)),
        ],
        out_shape=[
            jax.ShapeDtypeStruct((B * N, HIDDEN), f32),
            jax.ShapeDtypeStruct((B, 2 * HEADS, N), f32),
        ],
    )(node_feat, W_node.T, a_sd)
    sd = sd.reshape(B, 2 * HEADS * N)

    ECH = 4096
    a_edge8 = jnp.concatenate([a_edge, jnp.zeros((HIDDEN, HEADS), f32)], axis=1)
    e, el = pl.pallas_call(
        _prep_edges_body,
        grid=(B, E // ECH),
        in_specs=[
            pl.BlockSpec((1, ECH, EDGE_DIM), lambda b, c: (b, c, 0)),
            pl.BlockSpec((EDGE_DIM, HIDDEN), lambda b, c: (0, 0)),
            pl.BlockSpec((HIDDEN, 2 * HEADS), lambda b, c: (0, 0)),
        ],
        out_specs=[
            pl.BlockSpec((1, ECH, HIDDEN), lambda b, c: (b, c, 0)),
            pl.BlockSpec((1, 2 * HEADS, ECH), lambda b, c: (b, 0, c)),
        ],
        out_shape=[
            jax.ShapeDtypeStruct((B, E, HIDDEN), f32),
            jax.ShapeDtypeStruct((B, 2 * HEADS, E), f32),
        ],
    )(edge_attr, W_edge.T, a_edge8)

    src_r = edge_index[0].reshape(NSUB, NCHUNK, CHUNK)
    dst_r = edge_index[1].reshape(NSUB, NCHUNK, CHUNK)

    agg, den = _sc_gat(h, sd, e, el, src_r, dst_r)

    out = pl.pallas_call(
        _finish_body,
        grid=(B,),
        in_specs=[
            pl.BlockSpec((1, N, HIDDEN), lambda b: (b, 0, 0)),
            pl.BlockSpec((1, 8, N), lambda b: (b, 0, 0)),
            pl.BlockSpec((1, N, HIDDEN), lambda b: (b, 0, 0)),
            pl.BlockSpec((1, HIDDEN), lambda b: (0, 0)),
            pl.BlockSpec((1, HIDDEN), lambda b: (0, 0)),
        ],
        out_specs=pl.BlockSpec((1, N, HIDDEN), lambda b: (b, 0, 0)),
        out_shape=jax.ShapeDtypeStruct((B, N, HIDDEN), f32),
    )(agg, den, node_feat, ln_gamma.reshape(1, HIDDEN), ln_beta.reshape(1, HIDDEN))
    return out


# speedup vs baseline: 22.6218x; 1.1885x over previous
import functools

import jax
import jax.numpy as jnp
from jax import lax
from jax.experimental import pallas as pl
from jax.experimental.pallas import tpu as pltpu
from jax.experimental.pallas import tpu_sc as plsc

B, N, E = 8, 1024, 32768
NODE_DIM, EDGE_DIM, HIDDEN, HEADS = 128, 16, 128, 4
HEAD_DIM = HIDDEN // HEADS

NSUB = 16
NCORE = 2
EPT = E // NSUB
CHUNK = 128
NCHUNK = EPT // CHUNK
ROWS = N // NSUB
BPC = B // NCORE
DTOT = HEADS * N
DSL = DTOT // NSUB


def _prep_nodes_body(nf_ref, wnt_ref, asd_ref, h_ref, sd_ref):
    h = jnp.dot(nf_ref[0], wnt_ref[...], preferred_element_type=jnp.float32)
    h_ref[...] = h
    sd_ref[0] = lax.dot_general(asd_ref[...], h, (((0,), (1,)), ((), ())),
                                preferred_element_type=jnp.float32)


def _prep_edges_body(eat_ref, wet_ref, ae_ref, e_ref, el_ref):
    e = lax.dot_general(eat_ref[0], wet_ref[...], (((0,), (0,)), ((), ())),
                        preferred_element_type=jnp.float32)
    e_ref[0] = e
    el_ref[0] = lax.dot_general(ae_ref[...], e, (((0,), (1,)), ((), ())),
                                preferred_element_type=jnp.float32)


def _finish_body(agg_ref, den_ref, nf_ref, g_ref, b_ref, o_ref):
    den = den_ref[0, :HEADS].T
    inv = 1.0 / jnp.where(den > 0, den, 1.0)
    invr = jnp.reshape(
        jnp.broadcast_to(inv[:, :, None], (N, HEADS, HEAD_DIM)), (N, HIDDEN))
    res = agg_ref[0] * invr + nf_ref[0]
    mean = jnp.mean(res, axis=1, keepdims=True)
    xc = res - mean
    var = jnp.mean(xc * xc, axis=1, keepdims=True)
    y = xc * lax.rsqrt(var + 1e-5) * g_ref[...] + b_ref[...]
    o_ref[0] = jnp.where(y > 0, y, jnp.exp(y) - 1.0)


_sc_mesh = plsc.VectorSubcoreMesh(core_axis_name="c", subcore_axis_name="s")


@functools.partial(
    pl.kernel,
    out_type=(
        jax.ShapeDtypeStruct((B, N, HIDDEN), jnp.float32),
        jax.ShapeDtypeStruct((B, 8, N), jnp.float32),
    ),
    mesh=_sc_mesh,
    compiler_params=pltpu.CompilerParams(needs_layout_passes=False),
    scratch_types=[
        pltpu.VMEM((NCHUNK, CHUNK), jnp.int32),
        pltpu.VMEM((NCHUNK, CHUNK), jnp.int32),
        pltpu.VMEM((CHUNK,), jnp.int32),
        pltpu.VMEM((CHUNK,), jnp.int32),
        pltpu.VMEM((N * 8,), jnp.float32),
        pltpu.VMEM((EPT * HEADS,), jnp.float32),
        pltpu.VMEM((EPT * HEADS + 16,), jnp.float32),
        pltpu.VMEM((DTOT,), jnp.float32),
        pltpu.VMEM((NSUB * DSL,), jnp.float32),
        pltpu.VMEM((DSL,), jnp.float32),
        pltpu.VMEM((CHUNK, HIDDEN), jnp.float32),
        pltpu.VMEM((CHUNK, HIDDEN), jnp.float32),
        pltpu.VMEM((ROWS, HIDDEN), jnp.float32),
        pltpu.VMEM_SHARED((N, HIDDEN), jnp.float32),
        pltpu.VMEM_SHARED((NSUB * DTOT,), jnp.float32),
        pltpu.SemaphoreType.DMA,
    ],
)
def _sc_gat(h_hbm, sd_hbm, e_hbm, el_hbm, src_hbm, dst_hbm,
            agg_hbm, den_hbm,
            src_c_v, dst_c_v, gidx_v, didx_v, sd_v, el_v, w_v, den_v,
            red_v, dsum_v, e_v, hg_v, z_v, agg_sh, den_all_sh, sem):
    cid = lax.axis_index("c")
    sid = lax.axis_index("s")

    pltpu.sync_copy(src_hbm.at[sid], src_c_v)
    pltpu.sync_copy(dst_hbm.at[sid], dst_c_v)

    def _zz(i, _):
        for j in range(HIDDEN // 16):
            z_v[i, pl.ds(j * 16, 16)] = jnp.zeros((16,), jnp.float32)
        return 0
    lax.fori_loop(0, ROWS, _zz, 0)

    def batch_body(bl, _):
        b = cid * BPC + bl

        pltpu.sync_copy(sd_hbm.at[b], sd_v)
        for h in range(HEADS):
            pltpu.sync_copy(el_hbm.at[b, h, pl.ds(sid * EPT, EPT)],
                            el_v.at[pl.ds(h * EPT, EPT)])

        def _zd(i, _):
            den_v[pl.ds(i * 16, 16)] = jnp.zeros((16,), jnp.float32)
            return 0
        lax.fori_loop(0, DTOT // 16, _zd, 0)
        pltpu.sync_copy(z_v, agg_sh.at[pl.ds(sid * ROWS, ROWS)])
        plsc.subcore_barrier()

        def phase_a(g, _):
            c = g // (CHUNK // 16)
            o = (g % (CHUNK // 16)) * 16
            src16 = src_c_v[c, pl.ds(o, 16)]
            dst16 = dst_c_v[c, pl.ds(o, 16)]
            k16 = g * 16 + lax.iota(jnp.int32, 16)
            for h in range(HEADS):
                sv = plsc.load_gather(sd_v, [src16 + h * N])
                dv = plsc.load_gather(sd_v, [dst16 + (HEADS + h) * N])
                ev = el_v[pl.ds(h * EPT + g * 16, 16)]
                l = sv + dv + ev
                l = jnp.where(l >= 0, l, l * jnp.float32(0.2))
                w = jnp.exp(l)
                plsc.store_scatter(w_v, [k16 * HEADS + h], w)
                plsc.addupdate_scatter(den_v, [dst16 + h * N], w)
            return 0
        lax.fori_loop(0, EPT // 16, phase_a, 0)

        def chunk_body(cb, _):
            pltpu.sync_copy(e_hbm.at[b, pl.ds(sid * EPT + cb * CHUNK, CHUNK)], e_v)
            base = b * N
            for j in range(CHUNK // 16):
                gidx_v[pl.ds(j * 16, 16)] = src_c_v[cb, pl.ds(j * 16, 16)] + base
                didx_v[pl.ds(j * 16, 16)] = dst_c_v[cb, pl.ds(j * 16, 16)]
            pltpu.async_copy(h_hbm.at[gidx_v], hg_v, sem).wait()

            def edge_body(k, _):
                wrow = w_v[pl.ds((cb * CHUNK + k) * HEADS, 16)]
                for h in range(HEADS):
                    wb = jnp.full((16,), wrow[h])
                    for j2 in range(HEAD_DIM // 16):
                        col = h * HEAD_DIM + j2 * 16
                        m = (hg_v[k, pl.ds(col, 16)] + e_v[k, pl.ds(col, 16)]) * wb
                        hg_v[k, pl.ds(col, 16)] = m
                return 0
            lax.fori_loop(0, CHUNK, edge_body, 0)

            pltpu.sync_copy(hg_v, agg_sh.at[didx_v], add=True)
            return 0
        lax.fori_loop(0, NCHUNK, chunk_body, 0)

        pltpu.sync_copy(den_v, den_all_sh.at[pl.ds(sid * DTOT, DTOT)])
        plsc.subcore_barrier()

        pltpu.sync_copy(agg_sh.at[pl.ds(sid * ROWS, ROWS)],
                        agg_hbm.at[b, pl.ds(sid * ROWS, ROWS)])
        for t in range(NSUB):
            pltpu.sync_copy(den_all_sh.at[pl.ds(t * DTOT + sid * DSL, DSL)],
                            red_v.at[pl.ds(t * DSL, DSL)])
        for j in range(DSL // 16):
            acc = red_v[pl.ds(j * 16, 16)]
            for t in range(1, NSUB):
                acc = acc + red_v[pl.ds(t * DSL + j * 16, 16)]
            dsum_v[pl.ds(j * 16, 16)] = acc
        pltpu.sync_copy(dsum_v,
                        den_hbm.at[b, sid // (N // DSL),
                                   pl.ds((sid % (N // DSL)) * DSL, DSL)])
        plsc.subcore_barrier()
        return 0

    lax.fori_loop(0, BPC, batch_body, 0)


def kernel(node_feat, edge_index, edge_attr, W_node, W_edge,
           att_src, att_dst, att_edge, ln_gamma, ln_beta):
    f32 = jnp.float32
    eye = jnp.eye(HEADS, dtype=f32)
    a_src = (eye[:, None, :] * att_src[:, :, None]).reshape(HIDDEN, HEADS)
    a_dst = (eye[:, None, :] * att_dst[:, :, None]).reshape(HIDDEN, HEADS)
    a_edge = (eye[:, None, :] * att_edge[:, :, None]).reshape(HIDDEN, HEADS)
    a_sd = jnp.concatenate([a_src, a_dst], axis=1)

    h, sd = pl.pallas_call(
        _prep_nodes_body,
        grid=(B,),
        in_specs=[
            pl.BlockSpec((1, N, NODE_DIM), lambda b: (b, 0, 0)),
            pl.BlockSpec((NODE_DIM, HIDDEN), lambda b: (0, 0)),
            pl.BlockSpec((HIDDEN, 2 * HEADS), lambda b: (0, 0)),
        ],
        out_specs=[
            pl.BlockSpec((N, HIDDEN), lambda b: (b, 0)),
            pl.BlockSpec((1, 2 * HEADS, N), lambda b: (b, 0, 0)),
        ],
        out_shape=[
            jax.ShapeDtypeStruct((B * N, HIDDEN), f32),
            jax.ShapeDtypeStruct((B, 2 * HEADS, N), f32),
        ],
    )(node_feat, W_node.T, a_sd)
    sd = sd.reshape(B, 2 * HEADS * N)

    ECH = 4096
    a_edge8 = jnp.concatenate([a_edge, jnp.zeros((HIDDEN, HEADS), f32)], axis=1)
    edge_attr_t = jnp.transpose(edge_attr, (0, 2, 1))
    e, el = pl.pallas_call(
        _prep_edges_body,
        grid=(B, E // ECH),
        in_specs=[
            pl.BlockSpec((1, EDGE_DIM, ECH), lambda b, c: (b, 0, c)),
            pl.BlockSpec((EDGE_DIM, HIDDEN), lambda b, c: (0, 0)),
            pl.BlockSpec((HIDDEN, 2 * HEADS), lambda b, c: (0, 0)),
        ],
        out_specs=[
            pl.BlockSpec((1, ECH, HIDDEN), lambda b, c: (b, c, 0)),
            pl.BlockSpec((1, 2 * HEADS, ECH), lambda b, c: (b, 0, c)),
        ],
        out_shape=[
            jax.ShapeDtypeStruct((B, E, HIDDEN), f32),
            jax.ShapeDtypeStruct((B, 2 * HEADS, E), f32),
        ],
    )(edge_attr_t, W_edge.T, a_edge8)

    src_r = edge_index[0].reshape(NSUB, NCHUNK, CHUNK)
    dst_r = edge_index[1].reshape(NSUB, NCHUNK, CHUNK)

    agg, den = _sc_gat(h, sd, e, el, src_r, dst_r)

    out = pl.pallas_call(
        _finish_body,
        grid=(B,),
        in_specs=[
            pl.BlockSpec((1, N, HIDDEN), lambda b: (b, 0, 0)),
            pl.BlockSpec((1, 8, N), lambda b: (b, 0, 0)),
            pl.BlockSpec((1, N, HIDDEN), lambda b: (b, 0, 0)),
            pl.BlockSpec((1, HIDDEN), lambda b: (0, 0)),
            pl.BlockSpec((1, HIDDEN), lambda b: (0, 0)),
        ],
        out_specs=pl.BlockSpec((1, N, HIDDEN), lambda b: (b, 0, 0)),
        out_shape=jax.ShapeDtypeStruct((B, N, HIDDEN), f32),
    )(agg, den, node_feat, ln_gamma.reshape(1, HIDDEN), ln_beta.reshape(1, HIDDEN))
    return out
